# Initial kernel scaffold; baseline (speedup 1.0000x reference)
import functools

import jax
import jax.numpy as jnp
from jax import lax
from jax.experimental import pallas as pl
from jax.experimental.pallas import tpu as pltpu
from jax.experimental.pallas import tpu_sc as plsc

N = 10000
E = 160000
NUM_GATE = 29
D = 256
HD = 128
NS = 16
EPT = E // NS
K = 80
NCHUNK = EPT // K
NPT = N // NS
ZROWS = 25
BN = 2000


def _segsum_body(zn_a_lo, zn_a_hi, zn_c_lo, zn_c_hi, src_hbm, dst_hbm,
                 out_a_lo, out_a_hi, out_c_lo, out_c_hi,
                 srcbuf, dstbuf, rows, zbuf, shared, sem0, sem1):
    c = lax.axis_index("c")
    s = lax.axis_index("s")

    pltpu.sync_copy(src_hbm.at[s], srcbuf)
    pltpu.sync_copy(dst_hbm.at[s], dstbuf)

    z16 = jnp.zeros((16,), jnp.float32)
    for r in range(ZROWS):
        for l2 in range(HD // 16):
            zbuf[r, pl.ds(l2 * 16, 16)] = z16

    sems = (sem0, sem1)

    def zero_shared():
        base = s * NPT
        for j in range(NPT // ZROWS):
            pltpu.sync_copy(zbuf, shared.at[pl.ds(base + j * ZROWS, ZROWS)])

    def accumulate(zn_half):
        def gather(i, slot):
            idx = srcbuf.at[pl.ds(i * K, K)]
            pltpu.async_copy(zn_half.at[idx], rows.at[slot], sems[slot])

        def gwait(i, slot):
            idx = srcbuf.at[pl.ds(i * K, K)]
            pltpu.make_async_copy(zn_half.at[idx], rows.at[slot],
                                  sems[slot]).wait()

        gather(0, 0)

        def step(i, slot, nslot):
            @pl.when(i + 1 < NCHUNK)
            def _():
                gather(i + 1, nslot)
            gwait(i, slot)
            pltpu.sync_copy(rows.at[slot], shared.at[dstbuf.at[i]], add=True)

        def body(i, carry):
            @pl.when(lax.rem(i, 2) == 0)
            def _():
                step(i, 0, 1)

            @pl.when(lax.rem(i, 2) != 0)
            def _():
                step(i, 1, 0)
            return carry

        lax.fori_loop(0, NCHUNK, body, 0)

    def writeout(out_half):
        sl = pl.ds(s * NPT, NPT)
        pltpu.sync_copy(shared.at[sl], out_half.at[sl])

    def run(zn_a_h, zn_c_h, out_a_h, out_c_h):
        zero_shared()
        plsc.subcore_barrier()
        accumulate(zn_a_h)
        plsc.subcore_barrier()
        writeout(out_a_h)
        zero_shared()
        plsc.subcore_barrier()
        accumulate(zn_c_h)
        plsc.subcore_barrier()
        writeout(out_c_h)

    @pl.when(c == 0)
    def _():
        run(zn_a_lo, zn_c_lo, out_a_lo, out_c_lo)

    @pl.when(c == 1)
    def _():
        run(zn_a_hi, zn_c_hi, out_a_hi, out_c_hi)


_segsum = functools.partial(
    pl.kernel,
    out_type=[jax.ShapeDtypeStruct((N, HD), jnp.float32)] * 4,
    mesh=plsc.VectorSubcoreMesh(core_axis_name="c", subcore_axis_name="s"),
    scratch_types=[
        pltpu.VMEM((EPT,), jnp.int32),
        pltpu.VMEM((NCHUNK, K), jnp.int32),
        pltpu.VMEM((2, K, HD), jnp.float32),
        pltpu.VMEM((ZROWS, HD), jnp.float32),
        pltpu.VMEM_SHARED((N, HD), jnp.float32),
        pltpu.SemaphoreType.DMA,
        pltpu.SemaphoreType.DMA,
    ],
)(_segsum_body)


def _row_spec(w):
    return pl.BlockSpec((BN, w), lambda i: (i, 0))


def _full_spec(r, w):
    return pl.BlockSpec((r, w), lambda i: (0, 0))


def _split_z(z, o_zs, o_znl, o_znh):
    o_zs[...] = z[:, :D]
    o_znl[...] = z[:, D:D + HD]
    o_znh[...] = z[:, D + HD:]


def _l0_body(g, Wa, ba, Wc, bc,
             o_zs_a, o_znl_a, o_znh_a, o_zs_c, o_znl_c, o_znh_c):
    gi = g[...]
    iot = lax.broadcasted_iota(jnp.int32, (1, 32), 1)
    oh = (gi == iot).astype(jnp.float32)
    za = jnp.dot(oh, Wa[...], preferred_element_type=jnp.float32) + ba[...]
    zc = jnp.dot(oh, Wc[...], preferred_element_type=jnp.float32) + bc[...]
    _split_z(za, o_zs_a, o_znl_a, o_znh_a)
    _split_z(zc, o_zs_c, o_znl_c, o_znh_c)


def _fuse_body(zs_a, agl_a, agh_a, Wa, ba, zs_c, agl_c, agh_c, Wc, bc,
               o_zs_a, o_znl_a, o_znh_a, o_zs_c, o_znl_c, o_znh_c):
    def one(zs, agl, agh, W, b, o_zs, o_znl, o_znh):
        h = jnp.maximum(
            zs[...] + jnp.concatenate([agl[...], agh[...]], axis=1), 0.0)
        z = jnp.dot(h, W[...], preferred_element_type=jnp.float32) + b[...]
        _split_z(z, o_zs, o_znl, o_znh)

    one(zs_a, agl_a, agh_a, Wa, ba, o_zs_a, o_znl_a, o_znh_a)
    one(zs_c, agl_c, agh_c, Wc, bc, o_zs_c, o_znl_c, o_znh_c)


def _head_body(zs_a, agl_a, agh_a, zs_c, agl_c, agh_c,
               anW1, anb1, anW2, anb2, axW1, axb1, axW2, axb2,
               crW1, crb1, crW2, crb2, out):
    h_a = jnp.maximum(
        zs_a[...] + jnp.concatenate([agl_a[...], agh_a[...]], axis=1), 0.0)
    h_c = jnp.maximum(
        zs_c[...] + jnp.concatenate([agl_c[...], agh_c[...]], axis=1), 0.0)
    dot = lambda x, w: jnp.dot(x, w, preferred_element_type=jnp.float32)
    t1 = jnp.maximum(dot(h_a, anW1[...]) + anb1[...], 0.0)
    nl = dot(t1, anW2[...]) + anb2[...]
    t2 = jnp.maximum(dot(h_a, axW1[...]) + axb1[...], 0.0)
    xf = dot(t2, axW2[...]) + axb2[...]
    t3 = jnp.maximum(dot(h_c, crW1[...]) + crb1[...], 0.0)
    vs = dot(t3, crW2[...]) + crb2[...]
    out[...] = jnp.concatenate([nl, xf, vs], axis=1)


def _build_tc(interpret=False):
    grid = (N // BN,)
    z_outs = [
        jax.ShapeDtypeStruct((N, D), jnp.float32),
        jax.ShapeDtypeStruct((N, HD), jnp.float32),
        jax.ShapeDtypeStruct((N, HD), jnp.float32),
    ] * 2
    z_out_specs = [_row_spec(D), _row_spec(HD), _row_spec(HD)] * 2

    l0 = pl.pallas_call(
        _l0_body,
        grid=grid,
        in_specs=[_row_spec(1), _full_spec(32, 2 * D), _full_spec(1, 2 * D),
                  _full_spec(32, 2 * D), _full_spec(1, 2 * D)],
        out_specs=z_out_specs,
        out_shape=z_outs,
        interpret=interpret,
    )

    branch_in = [_row_spec(D), _row_spec(HD), _row_spec(HD),
                 _full_spec(D, 2 * D), _full_spec(1, 2 * D)]
    fuse = pl.pallas_call(
        _fuse_body,
        grid=grid,
        in_specs=branch_in * 2,
        out_specs=z_out_specs,
        out_shape=z_outs,
        interpret=interpret,
    )

    head = pl.pallas_call(
        _head_body,
        grid=grid,
        in_specs=[_row_spec(D), _row_spec(HD), _row_spec(HD)] * 2 + [
            _full_spec(D, HD), _full_spec(1, HD),
            _full_spec(HD, 1), _full_spec(1, 1),
            _full_spec(D, D), _full_spec(1, D),
            _full_spec(D, HD), _full_spec(1, HD),
            _full_spec(D, D), _full_spec(1, D),
            _full_spec(D, 1), _full_spec(1, 1),
        ],
        out_specs=_row_spec(130),
        out_shape=jax.ShapeDtypeStruct((N, 130), jnp.float32),
        interpret=interpret,
    )
    return l0, fuse, head


_L0, _FUSE, _HEAD = _build_tc()


def _layer_weights(lyr, pad_to=None):
    W = jnp.concatenate([lyr['Wself'], lyr['Wneigh']], axis=1)
    if pad_to is not None:
        W = jnp.pad(W, ((0, pad_to - W.shape[0]), (0, 0)))
    b2 = jnp.concatenate([lyr['b'], jnp.zeros((D,), jnp.float32)])
    return W, b2.reshape(1, 2 * D)


def _forward(gate_types, edge_index, params, l0, fuse, head, segsum):
    src2 = edge_index[0].reshape(NS, EPT)
    dst3 = edge_index[1].reshape(NS, NCHUNK, K)
    g2 = gate_types.reshape(N, 1)

    Wa0, ba0 = _layer_weights(params['actor_gnn'][0], pad_to=32)
    Wc0, bc0 = _layer_weights(params['critic_gnn'][0], pad_to=32)

    zs_a, znl_a, znh_a, zs_c, znl_c, znh_c = l0(g2, Wa0, ba0, Wc0, bc0)

    for i in range(1, 6):
        agl_a, agh_a, agl_c, agh_c = segsum(znl_a, znh_a, znl_c, znh_c,
                                            src2, dst3)
        Wa, ba = _layer_weights(params['actor_gnn'][i])
        Wc, bc = _layer_weights(params['critic_gnn'][i])
        zs_a, znl_a, znh_a, zs_c, znl_c, znh_c = fuse(
            zs_a, agl_a, agh_a, Wa, ba, zs_c, agl_c, agh_c, Wc, bc)

    agl_a, agh_a, agl_c, agh_c = segsum(znl_a, znh_a, znl_c, znh_c,
                                        src2, dst3)
    p = params
    return head(
        zs_a, agl_a, agh_a, zs_c, agl_c, agh_c,
        p['an_W1'], p['an_b1'].reshape(1, HD),
        p['an_W2'], p['an_b2'].reshape(1, 1),
        p['ax_W1'], p['ax_b1'].reshape(1, D),
        p['ax_W2'], p['ax_b2'].reshape(1, HD),
        p['cr_W1'], p['cr_b1'].reshape(1, D),
        p['cr_W2'], p['cr_b2'].reshape(1, 1),
    )


def kernel(gate_types, edge_index, params):
    return _forward(gate_types, edge_index, params, _L0, _FUSE, _HEAD,
                    _segsum)


# speedup vs baseline: 4.9560x; 4.9560x over previous
import functools

import jax
import jax.numpy as jnp
from jax import lax
from jax.experimental import pallas as pl
from jax.experimental.pallas import tpu as pltpu
from jax.experimental.pallas import tpu_sc as plsc

N = 10000
E = 160000
NUM_GATE = 29
D = 256
HD = 128
NS = 16
EPT = E // NS
K = 80
NCHUNK = EPT // K
NPT = 624
ZROWS = 16
BN = 2000


def _segsum_body(zn_a_lo, zn_a_hi, zn_c_lo, zn_c_hi, src_hbm, dst_hbm,
                 out_a_lo, out_a_hi, out_c_lo, out_c_hi,
                 srcbuf, dstbuf, rows, zbuf, shared, sem0, sem1):
    c = lax.axis_index("c")
    s = lax.axis_index("s")

    pltpu.sync_copy(src_hbm.at[s], srcbuf)
    pltpu.sync_copy(dst_hbm.at[s], dstbuf)

    z16 = jnp.zeros((16,), jnp.float32)
    for r in range(ZROWS):
        for l2 in range(HD // 16):
            zbuf[r, pl.ds(l2 * 16, 16)] = z16

    sems = (sem0, sem1)

    def zero_shared():
        base = s * NPT
        for j in range(NPT // ZROWS):
            pltpu.sync_copy(zbuf, shared.at[pl.ds(base + j * ZROWS, ZROWS)])

        @pl.when(s == NS - 1)
        def _():
            pltpu.sync_copy(zbuf, shared.at[pl.ds(NS * NPT, N - NS * NPT)])

    def accumulate(zn_half):
        def gather(i, slot):
            idx = srcbuf.at[pl.ds(i * K, K)]
            pltpu.async_copy(zn_half.at[idx], rows.at[slot], sems[slot])

        def gwait(i, slot):
            idx = srcbuf.at[pl.ds(i * K, K)]
            pltpu.make_async_copy(zn_half.at[idx], rows.at[slot],
                                  sems[slot]).wait()

        gather(0, 0)

        def step(i, slot, nslot):
            @pl.when(i + 1 < NCHUNK)
            def _():
                gather(i + 1, nslot)
            gwait(i, slot)
            pltpu.sync_copy(rows.at[slot], shared.at[dstbuf.at[i]], add=True)

        def body(i, carry):
            @pl.when(lax.rem(i, 2) == 0)
            def _():
                step(i, 0, 1)

            @pl.when(lax.rem(i, 2) != 0)
            def _():
                step(i, 1, 0)
            return carry

        lax.fori_loop(0, NCHUNK, body, 0)

    def writeout(out_half):
        sl = pl.ds(s * NPT, NPT)
        pltpu.sync_copy(shared.at[sl], out_half.at[sl])

        @pl.when(s == NS - 1)
        def _():
            tail = pl.ds(NS * NPT, N - NS * NPT)
            pltpu.sync_copy(shared.at[tail], out_half.at[tail])

    def run(zn_a_h, zn_c_h, out_a_h, out_c_h):
        zero_shared()
        plsc.subcore_barrier()
        accumulate(zn_a_h)
        plsc.subcore_barrier()
        writeout(out_a_h)
        zero_shared()
        plsc.subcore_barrier()
        accumulate(zn_c_h)
        plsc.subcore_barrier()
        writeout(out_c_h)

    @pl.when(c == 0)
    def _():
        run(zn_a_lo, zn_c_lo, out_a_lo, out_c_lo)

    @pl.when(c == 1)
    def _():
        run(zn_a_hi, zn_c_hi, out_a_hi, out_c_hi)


@functools.cache
def _get_segsum():
    return pl.kernel(
        _segsum_body,
        out_type=[jax.ShapeDtypeStruct((N, HD), jnp.float32)] * 4,
        mesh=plsc.VectorSubcoreMesh(core_axis_name="c", subcore_axis_name="s",
                                    num_cores=2, num_subcores=NS),
        scratch_types=[
            pltpu.VMEM((EPT,), jnp.int32),
            pltpu.VMEM((NCHUNK, K), jnp.int32),
            pltpu.VMEM((2, K, HD), jnp.float32),
            pltpu.VMEM((ZROWS, HD), jnp.float32),
            pltpu.VMEM_SHARED((N, HD), jnp.float32),
            pltpu.SemaphoreType.DMA,
            pltpu.SemaphoreType.DMA,
        ],
    )


def _row_spec(w):
    return pl.BlockSpec((BN, w), lambda i: (i, 0))


def _full_spec(r, w):
    return pl.BlockSpec((r, w), lambda i: (0, 0))


def _split_z(z, o_zs, o_znl, o_znh):
    o_zs[...] = z[:, :D]
    o_znl[...] = z[:, D:D + HD]
    o_znh[...] = z[:, D + HD:]


def _l0_body(g, Wa, ba, Wc, bc,
             o_zs_a, o_znl_a, o_znh_a, o_zs_c, o_znl_c, o_znh_c):
    gi = g[...]
    iot = lax.broadcasted_iota(jnp.int32, (1, 32), 1)
    oh = (gi == iot).astype(jnp.float32)
    za = jnp.dot(oh, Wa[...], preferred_element_type=jnp.float32) + ba[...]
    zc = jnp.dot(oh, Wc[...], preferred_element_type=jnp.float32) + bc[...]
    _split_z(za, o_zs_a, o_znl_a, o_znh_a)
    _split_z(zc, o_zs_c, o_znl_c, o_znh_c)


def _fuse_body(zs_a, agl_a, agh_a, Wa, ba, zs_c, agl_c, agh_c, Wc, bc,
               o_zs_a, o_znl_a, o_znh_a, o_zs_c, o_znl_c, o_znh_c):
    def one(zs, agl, agh, W, b, o_zs, o_znl, o_znh):
        h = jnp.maximum(
            zs[...] + jnp.concatenate([agl[...], agh[...]], axis=1), 0.0)
        z = jnp.dot(h, W[...], preferred_element_type=jnp.float32) + b[...]
        _split_z(z, o_zs, o_znl, o_znh)

    one(zs_a, agl_a, agh_a, Wa, ba, o_zs_a, o_znl_a, o_znh_a)
    one(zs_c, agl_c, agh_c, Wc, bc, o_zs_c, o_znl_c, o_znh_c)


def _head_body(zs_a, agl_a, agh_a, zs_c, agl_c, agh_c,
               anW1, anb1, anW2, anb2, axW1, axb1, axW2, axb2,
               crW1, crb1, crW2, crb2, out):
    h_a = jnp.maximum(
        zs_a[...] + jnp.concatenate([agl_a[...], agh_a[...]], axis=1), 0.0)
    h_c = jnp.maximum(
        zs_c[...] + jnp.concatenate([agl_c[...], agh_c[...]], axis=1), 0.0)
    dot = lambda x, w: jnp.dot(x, w, preferred_element_type=jnp.float32)
    t1 = jnp.maximum(dot(h_a, anW1[...]) + anb1[...], 0.0)
    nl = dot(t1, anW2[...]) + anb2[...]
    t2 = jnp.maximum(dot(h_a, axW1[...]) + axb1[...], 0.0)
    xf = dot(t2, axW2[...]) + axb2[...]
    t3 = jnp.maximum(dot(h_c, crW1[...]) + crb1[...], 0.0)
    vs = dot(t3, crW2[...]) + crb2[...]
    out[...] = jnp.concatenate([nl, xf, vs], axis=1)


def _build_tc(interpret=False):
    grid = (N // BN,)
    z_outs = [
        jax.ShapeDtypeStruct((N, D), jnp.float32),
        jax.ShapeDtypeStruct((N, HD), jnp.float32),
        jax.ShapeDtypeStruct((N, HD), jnp.float32),
    ] * 2
    z_out_specs = [_row_spec(D), _row_spec(HD), _row_spec(HD)] * 2

    l0 = pl.pallas_call(
        _l0_body,
        grid=grid,
        in_specs=[_row_spec(1), _full_spec(32, 2 * D), _full_spec(1, 2 * D),
                  _full_spec(32, 2 * D), _full_spec(1, 2 * D)],
        out_specs=z_out_specs,
        out_shape=z_outs,
        interpret=interpret,
    )

    branch_in = [_row_spec(D), _row_spec(HD), _row_spec(HD),
                 _full_spec(D, 2 * D), _full_spec(1, 2 * D)]
    fuse = pl.pallas_call(
        _fuse_body,
        grid=grid,
        in_specs=branch_in * 2,
        out_specs=z_out_specs,
        out_shape=z_outs,
        interpret=interpret,
    )

    head = pl.pallas_call(
        _head_body,
        grid=grid,
        in_specs=[_row_spec(D), _row_spec(HD), _row_spec(HD)] * 2 + [
            _full_spec(D, HD), _full_spec(1, HD),
            _full_spec(HD, 1), _full_spec(1, 1),
            _full_spec(D, D), _full_spec(1, D),
            _full_spec(D, HD), _full_spec(1, HD),
            _full_spec(D, D), _full_spec(1, D),
            _full_spec(D, 1), _full_spec(1, 1),
        ],
        out_specs=_row_spec(130),
        out_shape=jax.ShapeDtypeStruct((N, 130), jnp.float32),
        interpret=interpret,
    )
    return l0, fuse, head


_L0, _FUSE, _HEAD = _build_tc()


def _layer_weights(lyr, pad_to=None):
    W = jnp.concatenate([lyr['Wself'], lyr['Wneigh']], axis=1)
    if pad_to is not None:
        W = jnp.pad(W, ((0, pad_to - W.shape[0]), (0, 0)))
    b2 = jnp.concatenate([lyr['b'], jnp.zeros((D,), jnp.float32)])
    return W, b2.reshape(1, 2 * D)


def _forward(gate_types, edge_index, params, l0, fuse, head, segsum):
    src2 = edge_index[0].reshape(NS, EPT)
    dst3 = edge_index[1].reshape(NS, NCHUNK, K)
    g2 = gate_types.reshape(N, 1)

    Wa0, ba0 = _layer_weights(params['actor_gnn'][0], pad_to=32)
    Wc0, bc0 = _layer_weights(params['critic_gnn'][0], pad_to=32)

    zs_a, znl_a, znh_a, zs_c, znl_c, znh_c = l0(g2, Wa0, ba0, Wc0, bc0)

    for i in range(1, 6):
        agl_a, agh_a, agl_c, agh_c = segsum(znl_a, znh_a, znl_c, znh_c,
                                            src2, dst3)
        Wa, ba = _layer_weights(params['actor_gnn'][i])
        Wc, bc = _layer_weights(params['critic_gnn'][i])
        zs_a, znl_a, znh_a, zs_c, znl_c, znh_c = fuse(
            zs_a, agl_a, agh_a, Wa, ba, zs_c, agl_c, agh_c, Wc, bc)

    agl_a, agh_a, agl_c, agh_c = segsum(znl_a, znh_a, znl_c, znh_c,
                                        src2, dst3)
    p = params
    return head(
        zs_a, agl_a, agh_a, zs_c, agl_c, agh_c,
        p['an_W1'], p['an_b1'].reshape(1, HD),
        p['an_W2'], p['an_b2'].reshape(1, 1),
        p['ax_W1'], p['ax_b1'].reshape(1, D),
        p['ax_W2'], p['ax_b2'].reshape(1, HD),
        p['cr_W1'], p['cr_b1'].reshape(1, D),
        p['cr_W2'], p['cr_b2'].reshape(1, 1),
    )


def kernel(gate_types, edge_index, params):
    return _forward(gate_types, edge_index, params, _L0, _FUSE, _HEAD,
                    _get_segsum())
